# baseline (speedup 1.0000x reference)
import jax
import jax.numpy as jnp
from jax import lax
from jax.experimental import pallas as pl
from jax.experimental.pallas import tpu as pltpu

_NCLS = 8
_BLK = 8192


def _body(lab_ref, scale_ref, shift_ref, x_ref, o_ref):
    lab = lab_ref[...]
    iot = lax.broadcasted_iota(jnp.int32, (1, _NCLS), 1)
    onehot = (lab == iot).astype(jnp.float32)
    rs = jnp.dot(onehot, scale_ref[...], preferred_element_type=jnp.float32)
    rb = jnp.dot(onehot, shift_ref[...], preferred_element_type=jnp.float32)
    o_ref[...] = x_ref[...] * rs + rb


def kernel(x, labels, scale, shift):
    n, d = x.shape
    lab2 = labels.reshape(n, 1)
    grid = (n // _BLK,)
    return pl.pallas_call(
        _body,
        grid=grid,
        in_specs=[
            pl.BlockSpec((_BLK, 1), lambda i: (i, 0)),
            pl.BlockSpec((_NCLS, d), lambda i: (0, 0)),
            pl.BlockSpec((_NCLS, d), lambda i: (0, 0)),
            pl.BlockSpec((_BLK, d), lambda i: (i, 0)),
        ],
        out_specs=pl.BlockSpec((_BLK, d), lambda i: (i, 0)),
        out_shape=jax.ShapeDtypeStruct((n, d), x.dtype),
        compiler_params=pltpu.CompilerParams(
            dimension_semantics=("arbitrary",),
        ),
    )(lab2, scale, shift, x)
